# baseline (speedup 1.0000x reference)
import functools

import jax
import jax.numpy as jnp
from jax import lax
from jax.experimental import pallas as pl
from jax.experimental.pallas import tpu as pltpu
from jax.experimental.pallas import tpu_sc as plsc

VOCAB = 100000
DIM1 = 64
DIM2 = 128
DIM = DIM1 + DIM2
BATCH = 4096
SEQ = 200

NUM_CORES = 2
NUM_SUBCORES = 16
NW = NUM_CORES * NUM_SUBCORES
ROWS_PER_W = BATCH // NW

H0 = 104
H1 = SEQ - H0
HSIZE = (H0, H1)

RG = 4
NGRP = ROWS_PER_W // RG
TBODY = NGRP // 2
LANES = 16


def _make_kernel():
    mesh = plsc.VectorSubcoreMesh(core_axis_name="c", subcore_axis_name="s")

    @functools.partial(
        pl.kernel,
        mesh=mesh,
        out_type=(
            jax.ShapeDtypeStruct((BATCH, SEQ, DIM2), jnp.float32),
            jax.ShapeDtypeStruct((BATCH, SEQ, DIM1), jnp.float32),
        ),
        scratch_types=[
            pltpu.VMEM((RG, H0), jnp.int32),
            pltpu.VMEM((RG, H0), jnp.int32),
            pltpu.VMEM((RG, H1), jnp.int32),
            pltpu.VMEM((RG, H1), jnp.int32),
            pltpu.VMEM((H0, DIM2), jnp.float32),
            pltpu.VMEM((H1, DIM2), jnp.float32),
            pltpu.VMEM((H0, DIM2), jnp.float32),
            pltpu.VMEM((H1, DIM2), jnp.float32),
            pltpu.VMEM((H0, DIM1), jnp.float32),
            pltpu.VMEM((H1, DIM1), jnp.float32),
            pltpu.SemaphoreType.DMA,
            pltpu.SemaphoreType.DMA,
            pltpu.SemaphoreType.DMA,
            pltpu.SemaphoreType.DMA,
        ],
    )
    def stack_embed(
        wga_hbm,
        wgb_hbm,
        ta_hbm,
        t2_hbm,
        o1_hbm,
        o2_hbm,
        idxa0,
        idxa1,
        idxb0,
        idxb1,
        comb0,
        comb1,
        r20,
        r21,
        tail0,
        tail1,
        sem_g0,
        sem_g1,
        sem_i0,
        sem_i1,
    ):
        wid = lax.axis_index("s") * NUM_CORES + lax.axis_index("c")
        base_row = wid * ROWS_PER_W
        idxa = (idxa0, idxa1)
        idxb = (idxb0, idxb1)
        comb = (comb0, comb1)
        r2 = (r20, r21)
        tail = (tail0, tail1)
        sem_g = (sem_g0, sem_g1)
        sem_i = (sem_i0, sem_i1)

        def idx_row(p, r, h):
            return idxa[p].at[r] if h == 0 else idxb[p].at[r]

        def issue(p, r, h):
            pltpu.async_copy(ta_hbm.at[idx_row(p, r, h)], comb[h], sem_g[h])
            pltpu.async_copy(t2_hbm.at[idx_row(p, r, h)], r2[h], sem_g[h])

        def drain(p, r, h):
            pltpu.make_async_copy(
                ta_hbm.at[idx_row(p, r, h)], comb[h], sem_g[h]
            ).wait()
            pltpu.make_async_copy(
                t2_hbm.at[idx_row(p, r, h)], r2[h], sem_g[h]
            ).wait()

        def stage_idx(g, p):
            pltpu.async_copy(wga_hbm.at[wid, g], idxa[p], sem_i[p])
            pltpu.async_copy(wgb_hbm.at[wid, g], idxb[p], sem_i[p])

        def drain_idx(g, p):
            pltpu.make_async_copy(wga_hbm.at[wid, g], idxa[p], sem_i[p]).wait()
            pltpu.make_async_copy(wgb_hbm.at[wid, g], idxb[p], sem_i[p]).wait()

        def proc(g, p, r, h, issue_next):
            size = HSIZE[h]
            drain(p, r, h)
            issue_next()

            def tail2(jj, c):
                for rr in range(2):
                    row = jj * 2 + rr
                    for k in range(DIM1 // LANES):
                        tail[h][row, pl.ds(k * LANES, LANES)] = r2[h][
                            row, pl.ds(DIM1 + k * LANES, LANES)
                        ]
                return c

            lax.fori_loop(0, size // 2, tail2, 0)
            b = base_row + g * RG + r
            pltpu.sync_copy(comb[h], o1_hbm.at[b, pl.ds(h * H0, size)])
            pltpu.sync_copy(tail[h], o2_hbm.at[b, pl.ds(h * H0, size)])

        def group(g, p, t, last_issue):
            for r in range(RG):
                for h in range(2):
                    if h == 0:
                        nxt = lambda p=p, r=r: issue(p, r, 1)
                    elif r < RG - 1:
                        nxt = lambda p=p, r=r: issue(p, r + 1, 0)
                    else:
                        nxt = last_issue
                    proc(g, p, r, h, nxt)

        def body(t, carry):
            g0 = 2 * t
            g1 = g0 + 1
            stage_idx(g1, 1)

            def into_g1():
                drain_idx(g1, 1)
                issue(1, 0, 0)

            group(g0, 0, t, into_g1)

            @pl.when(t < TBODY - 1)
            def _():
                stage_idx(g0 + 2, 0)

            def into_next_body():
                @pl.when(t < TBODY - 1)
                def _():
                    drain_idx(g0 + 2, 0)
                    issue(0, 0, 0)

            group(g1, 1, t, into_next_body)
            return carry

        pltpu.sync_copy(wga_hbm.at[wid, 0], idxa[0])
        pltpu.sync_copy(wgb_hbm.at[wid, 0], idxb[0])
        issue(0, 0, 0)
        lax.fori_loop(0, TBODY, body, 0)

    return stack_embed


_STACK_EMBED = _make_kernel()


def kernel(words, table1, table2):
    ta = jnp.concatenate([table1, table2[:, :DIM1]], axis=1)
    wr = words.reshape(NW, ROWS_PER_W, SEQ).astype(jnp.int32)
    wga = wr[:, :, :H0].reshape(NW, NGRP, RG, H0)
    wgb = wr[:, :, H0:].reshape(NW, NGRP, RG, H1)
    o1, o2 = _STACK_EMBED(wga, wgb, ta, table2)
    return jnp.concatenate([o1, o2], axis=-1)


# speedup vs baseline: 1.0209x; 1.0209x over previous
import functools

import jax
import jax.numpy as jnp
from jax import lax
from jax.experimental import pallas as pl
from jax.experimental.pallas import tpu as pltpu
from jax.experimental.pallas import tpu_sc as plsc

VOCAB = 100000
DIM1 = 64
DIM2 = 128
DIM = DIM1 + DIM2
BATCH = 4096
SEQ = 200
N = BATCH * SEQ

NUM_CORES = 2
NUM_SUBCORES = 16
NW = NUM_CORES * NUM_SUBCORES
ROWS_PER_W = BATCH // NW

H0 = 104
H1 = SEQ - H0
HSIZE = (H0, H1)
P0 = 56
P1 = 48
PROW = P0 + P1
POFF = (0, P0)

RG = 4
NGRP = ROWS_PER_W // RG
TBODY = NGRP // 2
LANES = 16


def _make_kernel():
    mesh = plsc.VectorSubcoreMesh(core_axis_name="c", subcore_axis_name="s")

    @functools.partial(
        pl.kernel,
        mesh=mesh,
        out_type=(
            jax.ShapeDtypeStruct((N, DIM2), jnp.float32),
            jax.ShapeDtypeStruct((BATCH * PROW, DIM2), jnp.float32),
        ),
        scratch_types=[
            pltpu.VMEM((RG, H0), jnp.int32),
            pltpu.VMEM((RG, H0), jnp.int32),
            pltpu.VMEM((RG, H1), jnp.int32),
            pltpu.VMEM((RG, H1), jnp.int32),
            pltpu.VMEM((H0, DIM2), jnp.float32),
            pltpu.VMEM((H1, DIM2), jnp.float32),
            pltpu.VMEM((H0, DIM2), jnp.float32),
            pltpu.VMEM((H1, DIM2), jnp.float32),
            pltpu.VMEM((P0, DIM2), jnp.float32),
            pltpu.VMEM((P1, DIM2), jnp.float32),
            pltpu.SemaphoreType.DMA,
            pltpu.SemaphoreType.DMA,
            pltpu.SemaphoreType.DMA,
            pltpu.SemaphoreType.DMA,
        ],
    )
    def stack_embed(
        wga_hbm,
        wgb_hbm,
        ta_hbm,
        t2_hbm,
        o1_hbm,
        o2_hbm,
        idxa0,
        idxa1,
        idxb0,
        idxb1,
        comb0,
        comb1,
        r20,
        r21,
        tail0,
        tail1,
        sem_g0,
        sem_g1,
        sem_i0,
        sem_i1,
    ):
        wid = lax.axis_index("s") * NUM_CORES + lax.axis_index("c")
        base_row = wid * ROWS_PER_W
        idxa = (idxa0, idxa1)
        idxb = (idxb0, idxb1)
        comb = (comb0, comb1)
        r2 = (r20, r21)
        tail = (tail0, tail1)
        psize = (P0, P1)
        sem_g = (sem_g0, sem_g1)
        sem_i = (sem_i0, sem_i1)

        def idx_row(p, r, h):
            return idxa[p].at[r] if h == 0 else idxb[p].at[r]

        def issue(p, r, h):
            pltpu.async_copy(ta_hbm.at[idx_row(p, r, h)], comb[h], sem_g[h])
            pltpu.async_copy(t2_hbm.at[idx_row(p, r, h)], r2[h], sem_g[h])

        def drain(p, r, h):
            pltpu.make_async_copy(
                ta_hbm.at[idx_row(p, r, h)], comb[h], sem_g[h]
            ).wait()
            pltpu.make_async_copy(
                t2_hbm.at[idx_row(p, r, h)], r2[h], sem_g[h]
            ).wait()

        def stage_idx(g, p):
            pltpu.async_copy(wga_hbm.at[wid, g], idxa[p], sem_i[p])
            pltpu.async_copy(wgb_hbm.at[wid, g], idxb[p], sem_i[p])

        def drain_idx(g, p):
            pltpu.make_async_copy(wga_hbm.at[wid, g], idxa[p], sem_i[p]).wait()
            pltpu.make_async_copy(wgb_hbm.at[wid, g], idxb[p], sem_i[p]).wait()

        def proc(g, p, r, h, issue_next):
            size = HSIZE[h]
            drain(p, r, h)
            issue_next()

            def pack(q, c):
                for par in range(2):
                    for k in range(DIM1 // LANES):
                        tail[h][q, pl.ds(par * DIM1 + k * LANES, LANES)] = r2[
                            h
                        ][2 * q + par, pl.ds(DIM1 + k * LANES, LANES)]
                return c

            lax.fori_loop(0, size // 2, pack, 0)
            b = base_row + g * RG + r
            pltpu.sync_copy(comb[h], o1_hbm.at[pl.ds(b * SEQ + h * H0, size)])
            pltpu.sync_copy(
                tail[h], o2_hbm.at[pl.ds(b * PROW + POFF[h], psize[h])]
            )

        def group(g, p, t, last_issue):
            for r in range(RG):
                for h in range(2):
                    if h == 0:
                        nxt = lambda p=p, r=r: issue(p, r, 1)
                    elif r < RG - 1:
                        nxt = lambda p=p, r=r: issue(p, r + 1, 0)
                    else:
                        nxt = last_issue
                    proc(g, p, r, h, nxt)

        def body(t, carry):
            g0 = 2 * t
            g1 = g0 + 1
            stage_idx(g1, 1)

            def into_g1():
                drain_idx(g1, 1)
                issue(1, 0, 0)

            group(g0, 0, t, into_g1)

            @pl.when(t < TBODY - 1)
            def _():
                stage_idx(g0 + 2, 0)

            def into_next_body():
                @pl.when(t < TBODY - 1)
                def _():
                    drain_idx(g0 + 2, 0)
                    issue(0, 0, 0)

            group(g1, 1, t, into_next_body)
            return carry

        pltpu.sync_copy(wga_hbm.at[wid, 0], idxa[0])
        pltpu.sync_copy(wgb_hbm.at[wid, 0], idxb[0])
        issue(0, 0, 0)
        lax.fori_loop(0, TBODY, body, 0)

    return stack_embed


_STACK_EMBED = _make_kernel()


def kernel(words, table1, table2):
    ta = jnp.concatenate([table1, table2[:, :DIM1]], axis=1)
    wr = words.reshape(NW, ROWS_PER_W, SEQ).astype(jnp.int32)
    wga = wr[:, :, :H0].reshape(NW, NGRP, RG, H0)
    wgb = wr[:, :, H0:].reshape(NW, NGRP, RG, H1)
    o1, o2p = _STACK_EMBED(wga, wgb, ta, table2)
    head = o1.reshape(BATCH, SEQ, DIM2)
    o3 = o2p.reshape(BATCH, PROW, DIM2)
    tails_a = o3[:, : H0 // 2, :].reshape(BATCH, H0, DIM1)
    tails_b = o3[:, P0 : P0 + H1 // 2, :].reshape(BATCH, H1, DIM1)
    tails = jnp.concatenate([tails_a, tails_b], axis=1)
    return jnp.concatenate([head, tails], axis=-1)


# speedup vs baseline: 1.2239x; 1.1988x over previous
import functools

import jax
import jax.numpy as jnp
from jax import lax
from jax.experimental import pallas as pl
from jax.experimental.pallas import tpu as pltpu
from jax.experimental.pallas import tpu_sc as plsc

VOCAB = 100000
DIM1 = 64
DIM2 = 128
DIM = DIM1 + DIM2
BATCH = 4096
SEQ = 200

NUM_CORES = 2
NUM_SUBCORES = 16
NW = NUM_CORES * NUM_SUBCORES
ROWS_PER_W = BATCH // NW

H0 = 104
H1 = SEQ - H0
HSIZE = (H0, H1)

RG = 4
NGRP = ROWS_PER_W // RG
TBODY = NGRP // 2
LANES = 16


def _make_kernel():
    mesh = plsc.VectorSubcoreMesh(core_axis_name="c", subcore_axis_name="s")

    @functools.partial(
        pl.kernel,
        mesh=mesh,
        out_type=jax.ShapeDtypeStruct((BATCH, SEQ, DIM), jnp.float32),
        scratch_types=[
            pltpu.VMEM((RG, H0), jnp.int32),
            pltpu.VMEM((RG, H0), jnp.int32),
            pltpu.VMEM((RG, H1), jnp.int32),
            pltpu.VMEM((RG, H1), jnp.int32),
            pltpu.VMEM((H0, DIM), jnp.float32),
            pltpu.VMEM((H1, DIM), jnp.float32),
            pltpu.VMEM((H0, DIM2), jnp.float32),
            pltpu.VMEM((H1, DIM2), jnp.float32),
            pltpu.SemaphoreType.DMA,
            pltpu.SemaphoreType.DMA,
            pltpu.SemaphoreType.DMA,
            pltpu.SemaphoreType.DMA,
        ],
    )
    def stack_embed(
        wga_hbm,
        wgb_hbm,
        ta_hbm,
        t2_hbm,
        out_hbm,
        idxa0,
        idxa1,
        idxb0,
        idxb1,
        comb0,
        comb1,
        r20,
        r21,
        sem_g0,
        sem_g1,
        sem_i0,
        sem_i1,
    ):
        wid = lax.axis_index("s") * NUM_CORES + lax.axis_index("c")
        base_row = wid * ROWS_PER_W
        idxa = (idxa0, idxa1)
        idxb = (idxb0, idxb1)
        comb = (comb0, comb1)
        r2 = (r20, r21)
        sem_g = (sem_g0, sem_g1)
        sem_i = (sem_i0, sem_i1)

        def idx_row(p, r, h):
            return idxa[p].at[r] if h == 0 else idxb[p].at[r]

        def issue(p, r, h):
            pltpu.async_copy(
                ta_hbm.at[idx_row(p, r, h)], comb[h].at[:, pl.ds(0, DIM2)], sem_g[h]
            )
            pltpu.async_copy(t2_hbm.at[idx_row(p, r, h)], r2[h], sem_g[h])

        def drain(p, r, h):
            pltpu.make_async_copy(
                ta_hbm.at[idx_row(p, r, h)], comb[h].at[:, pl.ds(0, DIM2)], sem_g[h]
            ).wait()
            pltpu.make_async_copy(
                t2_hbm.at[idx_row(p, r, h)], r2[h], sem_g[h]
            ).wait()

        def stage_idx(g, p):
            pltpu.async_copy(wga_hbm.at[wid, g], idxa[p], sem_i[p])
            pltpu.async_copy(wgb_hbm.at[wid, g], idxb[p], sem_i[p])

        def drain_idx(g, p):
            pltpu.make_async_copy(wga_hbm.at[wid, g], idxa[p], sem_i[p]).wait()
            pltpu.make_async_copy(wgb_hbm.at[wid, g], idxb[p], sem_i[p]).wait()

        def proc(g, p, r, h, issue_next):
            size = HSIZE[h]
            drain(p, r, h)
            issue_next()

            def tail2(jj, c):
                for rr in range(2):
                    row = jj * 2 + rr
                    for k in range(DIM1 // LANES):
                        comb[h][row, pl.ds(DIM2 + k * LANES, LANES)] = r2[h][
                            row, pl.ds(DIM1 + k * LANES, LANES)
                        ]
                return c

            lax.fori_loop(0, size // 2, tail2, 0)
            b = base_row + g * RG + r
            pltpu.sync_copy(comb[h], out_hbm.at[b, pl.ds(h * H0, size)])

        def group(g, p, t, last_issue):
            for r in range(RG):
                for h in range(2):
                    if h == 0:
                        nxt = lambda p=p, r=r: issue(p, r, 1)
                    elif r < RG - 1:
                        nxt = lambda p=p, r=r: issue(p, r + 1, 0)
                    else:
                        nxt = last_issue
                    proc(g, p, r, h, nxt)

        def body(t, carry):
            g0 = 2 * t
            g1 = g0 + 1
            stage_idx(g1, 1)

            def into_g1():
                drain_idx(g1, 1)
                issue(1, 0, 0)

            group(g0, 0, t, into_g1)

            @pl.when(t < TBODY - 1)
            def _():
                stage_idx(g0 + 2, 0)

            def into_next_body():
                @pl.when(t < TBODY - 1)
                def _():
                    drain_idx(g0 + 2, 0)
                    issue(0, 0, 0)

            group(g1, 1, t, into_next_body)
            return carry

        pltpu.sync_copy(wga_hbm.at[wid, 0], idxa[0])
        pltpu.sync_copy(wgb_hbm.at[wid, 0], idxb[0])
        issue(0, 0, 0)
        lax.fori_loop(0, TBODY, body, 0)

    return stack_embed


_STACK_EMBED = _make_kernel()


def _noop_body(x_ref, o_ref):
    pass


_PIN_LAYOUT = pl.pallas_call(
    _noop_body,
    in_specs=[pl.BlockSpec(memory_space=pl.ANY)],
    out_specs=pl.BlockSpec(memory_space=pl.ANY),
    out_shape=jax.ShapeDtypeStruct((BATCH, SEQ, DIM), jnp.float32),
    input_output_aliases={0: 0},
)


def kernel(words, table1, table2):
    ta = jnp.concatenate([table1, table2[:, :DIM1]], axis=1)
    wr = words.reshape(NW, ROWS_PER_W, SEQ).astype(jnp.int32)
    wga = wr[:, :, :H0].reshape(NW, NGRP, RG, H0)
    wgb = wr[:, :, H0:].reshape(NW, NGRP, RG, H1)
    out = _STACK_EMBED(wga, wgb, ta, table2)
    return _PIN_LAYOUT(out)


# speedup vs baseline: 1.3913x; 1.1368x over previous
import functools

import jax
import jax.numpy as jnp
from jax import lax
from jax.experimental import pallas as pl
from jax.experimental.pallas import tpu as pltpu
from jax.experimental.pallas import tpu_sc as plsc

VOCAB = 100000
DIM1 = 64
DIM2 = 128
DIM = DIM1 + DIM2
BATCH = 4096
SEQ = 200
N = BATCH * SEQ

NUM_CORES = 2
NUM_SUBCORES = 16
NW = NUM_CORES * NUM_SUBCORES
PER_W = N // NW

CHUNK = 128
STEPS = PER_W // CHUNK
SUPER = 10
GROUPS = STEPS // SUPER
LANES = 16


def _make_kernel():
    mesh = plsc.VectorSubcoreMesh(core_axis_name="c", subcore_axis_name="s")

    @functools.partial(
        pl.kernel,
        mesh=mesh,
        out_type=jax.ShapeDtypeStruct((N // CHUNK, CHUNK, DIM), jnp.float32),
        scratch_types=[
            pltpu.VMEM((SUPER, CHUNK), jnp.int32),
            pltpu.VMEM((SUPER, CHUNK), jnp.int32),
            pltpu.VMEM((CHUNK, DIM), jnp.float32),
            pltpu.VMEM((CHUNK, DIM), jnp.float32),
            pltpu.VMEM((CHUNK, DIM2), jnp.float32),
            pltpu.VMEM((CHUNK, DIM2), jnp.float32),
            pltpu.SemaphoreType.DMA,
            pltpu.SemaphoreType.DMA,
            pltpu.SemaphoreType.DMA,
            pltpu.SemaphoreType.DMA,
        ],
    )
    def stack_embed(
        words_hbm,
        ta_hbm,
        t2_hbm,
        out_hbm,
        idx_a,
        idx_b,
        comb_a,
        comb_b,
        r2_a,
        r2_b,
        sem_ga,
        sem_gb,
        sem_ia,
        sem_ib,
    ):
        wid = lax.axis_index("s") * NUM_CORES + lax.axis_index("c")
        idx = (idx_a, idx_b)
        comb = (comb_a, comb_b)
        r2 = (r2_a, r2_b)
        sem_g = (sem_ga, sem_gb)
        sem_i = (sem_ia, sem_ib)

        def issue(idx_row, b):
            pltpu.async_copy(
                ta_hbm.at[idx_row], comb[b].at[:, pl.ds(0, DIM2)], sem_g[b]
            )
            pltpu.async_copy(t2_hbm.at[idx_row], r2[b], sem_g[b])

        def drain(idx_row, b):
            pltpu.make_async_copy(
                ta_hbm.at[idx_row], comb[b].at[:, pl.ds(0, DIM2)], sem_g[b]
            ).wait()
            pltpu.make_async_copy(t2_hbm.at[idx_row], r2[b], sem_g[b]).wait()

        def process(j, b, cur_row, issue_next):
            drain(cur_row, b)
            issue_next()

            def tail2(jj, c):
                for r in range(2):
                    row = jj * 2 + r
                    for k in range(DIM1 // LANES):
                        comb[b][row, pl.ds(DIM2 + k * LANES, LANES)] = r2[b][
                            row, pl.ds(DIM1 + k * LANES, LANES)
                        ]
                return c

            lax.fori_loop(0, CHUNK // 2, tail2, 0)
            pltpu.sync_copy(comb[b], out_hbm.at[wid * STEPS + j])

        def group(g, p, prefetch, t):
            prefetch()
            for c in range(SUPER):
                b = c % 2
                if c < SUPER - 1:
                    nxt = lambda c=c, b=b: issue(idx[p].at[c + 1], b ^ 1)
                elif p == 0:
                    def nxt(b=b):
                        pltpu.make_async_copy(
                            words_hbm.at[wid, 0], idx[1], sem_i[1]
                        ).wait()
                        issue(idx[1].at[0], b ^ 1)

                else:
                    def nxt(b=b):
                        @pl.when(t < GROUPS // 2 - 1)
                        def _():
                            pltpu.make_async_copy(
                                words_hbm.at[wid, 0], idx[0], sem_i[0]
                            ).wait()
                            issue(idx[0].at[0], b ^ 1)

                process(g * SUPER + c, b, idx[p].at[c], nxt)

        def body(t, carry):
            g0 = t * 2
            g1 = g0 + 1

            def pre0():
                pltpu.async_copy(words_hbm.at[wid, g1], idx[1], sem_i[1])

            def pre1():
                @pl.when(t < GROUPS // 2 - 1)
                def _():
                    pltpu.async_copy(words_hbm.at[wid, g0 + 2], idx[0], sem_i[0])

            group(g0, 0, pre0, t)
            group(g1, 1, pre1, t)
            return carry

        pltpu.sync_copy(words_hbm.at[wid, 0], idx[0])
        issue(idx[0].at[0], 0)
        lax.fori_loop(0, GROUPS // 2, body, 0)

    return stack_embed


_STACK_EMBED = _make_kernel()


def kernel(words, table1, table2):
    ta = jnp.concatenate([table1, table2[:, :DIM1]], axis=1)
    w4 = words.reshape(NW, GROUPS, SUPER, CHUNK).astype(jnp.int32)
    out = _STACK_EMBED(w4, ta, table2)
    return out.reshape(BATCH, SEQ, DIM)
